# Initial kernel scaffold; baseline (speedup 1.0000x reference)
import jax
import jax.numpy as jnp
from jax.experimental import pallas as pl


def kernel(X1, X2):
    raise NotImplementedError("write your pallas kernel here")


# speedup vs baseline: 4.7094x; 4.7094x over previous
import functools

import jax
import jax.numpy as jnp
import numpy as np
from jax.experimental import pallas as pl


def _compress(sequence, epsilon=1e-15):
    result = []
    for i in range(len(sequence)):
        m1, m2, _ = sequence[i]
        if any(m1 == r0 and m2 == r1 for r0, r1, _ in result):
            continue
        multiplier = sum(s[2] for s in sequence[i:] if s[0] == m1 and s[1] == m2)
        if np.abs(multiplier) > epsilon:
            result.append([m1, m2, multiplier])
    return result


def _get_conversion(l, m):
    if m < 0:
        X_re = [abs(m) + l, 1.0 / np.sqrt(2)]
        X_im = [m + l, -1.0 / np.sqrt(2)]
    if m == 0:
        X_re = [l, 1.0]
        X_im = [l, 0.0]
    if m > 0:
        if m % 2 == 0:
            X_re = [m + l, 1.0 / np.sqrt(2)]
            X_im = [-m + l, 1.0 / np.sqrt(2)]
        else:
            X_re = [m + l, -1.0 / np.sqrt(2)]
            X_im = [-m + l, -1.0 / np.sqrt(2)]
    return (X_re, X_im)


def _precompute_transformation(clebsch, l1, l2, lambd):
    def mul(first, second, multiplier):
        return [first[0], second[0], first[1] * second[1] * multiplier]

    def mul_seq(sequence, multiplier):
        return [[el[0], el[1], el[2] * multiplier] for el in sequence]

    result = [[] for _ in range(2 * lambd + 1)]
    for mu in range(0, lambd + 1):
        real_now = []
        imag_now = []
        for m2 in range(max(-l2, mu - l1), min(l2, mu + l1) + 1):
            m1 = mu - m2
            X1_re, X1_im = _get_conversion(l1, m1)
            X2_re, X2_im = _get_conversion(l2, m2)
            c = clebsch[m1 + l1, m2 + l2]
            real_now.append(mul(X1_re, X2_re, c))
            real_now.append(mul(X1_im, X2_im, -c))
            imag_now.append(mul(X1_re, X2_im, c))
            imag_now.append(mul(X1_im, X2_re, c))
        if (l1 + l2 - lambd) % 2 == 1:
            imag_now, real_now = (real_now, mul_seq(imag_now, -1))
        if mu > 0:
            s = np.sqrt(2) if mu % 2 == 0 else -np.sqrt(2)
            result[mu + lambd] = mul_seq(real_now, s)
            result[-mu + lambd] = mul_seq(imag_now, s)
        else:
            result[lambd] = real_now
    return [_compress(seq) for seq in result]


_L1 = 3
_L2 = 3
_LAMBD = 3
_NSLOT = 2 * _LAMBD + 1
_NSQ = _NSLOT * _NSLOT
_CLEBSCH = np.array(
    [[0.1 * (i + 1) + 0.01 * (j + 1) for j in range(2 * _L2 + 1)]
     for i in range(2 * _L1 + 1)], dtype=np.float64)
_TRANSF = _precompute_transformation(_CLEBSCH, _L1, _L2, _LAMBD)


def _build_w():
    w = np.zeros((_NSQ * _NSQ, _NSQ), dtype=np.float64)
    for mu in range(_NSLOT):
        for a1, a2, am in _TRANSF[mu]:
            for mup in range(_NSLOT):
                for b1, b2, bm in _TRANSF[mup]:
                    p = a1 * _NSLOT + b1
                    q = a2 * _NSLOT + b2
                    o = mu * _NSLOT + mup
                    w[p * _NSQ + q, o] += am * bm
    return w.astype(np.float32)


_W = jnp.asarray(_build_w())


def _wigner_block(x1_ref, x2_ref, w_ref, out_ref):
    x1 = x1_ref[...]
    x2 = x2_ref[...]
    outer = x1[:, :, None] * x2[:, None, :]
    outer = outer.reshape(x1.shape[0], _NSQ * _NSQ)
    out_ref[...] = jnp.dot(outer, w_ref[...],
                           preferred_element_type=jnp.float32)


@jax.jit
def kernel(X1, X2):
    batch = X1.shape[0]
    bc = 512
    x1f = X1.reshape(batch, _NSQ)
    x2f = X2.reshape(batch, _NSQ)
    out = pl.pallas_call(
        _wigner_block,
        grid=(batch // bc,),
        in_specs=[
            pl.BlockSpec((bc, _NSQ), lambda i: (i, 0)),
            pl.BlockSpec((bc, _NSQ), lambda i: (i, 0)),
            pl.BlockSpec((_NSQ * _NSQ, _NSQ), lambda i: (0, 0)),
        ],
        out_specs=pl.BlockSpec((bc, _NSQ), lambda i: (i, 0)),
        out_shape=jax.ShapeDtypeStruct((batch, _NSQ), jnp.float32),
    )(x1f, x2f, _W)
    return out.reshape(batch, _NSLOT, _NSLOT)
